# Initial kernel scaffold; baseline (speedup 1.0000x reference)
import jax
import jax.numpy as jnp
from jax.experimental import pallas as pl


def kernel(inputs, ref_img):
    raise NotImplementedError("write your pallas kernel here")


# speedup vs baseline: 1.1261x; 1.1261x over previous
import functools
import math

import jax
import jax.numpy as jnp
from jax import lax
from jax.experimental import pallas as pl
from jax.experimental.pallas import tpu as pltpu
from jax.experimental.pallas import tpu_sc as plsc

_H = 512
_W = 512
_B = 2
_IMH = 2048
_IMW = 2048
_NPOS = _B * _H * _W
_NPIX = _IMH * _IMW
_HW = _H * _W

_NW = 32
_PERW = _NPOS // _NW
_P = 2048
_CHUNKS = _PERW // _P

_PI = math.pi


def _coord_body(in_ref, idx_ref, w_ref):
    a0 = -in_ref[0, 0]
    a1 = -in_ref[0, 1]
    a2 = -in_ref[0, 2]
    a3 = -in_ref[0, 3]
    ty = jnp.arctan2(a1, a0)
    tx = jnp.arctan2(a3, a2)

    def to_px(t):
        ic = (t + _PI) / (2.0 * _PI)
        ic = -1.0 + 2.0 * ic
        return (ic + 1.0) * 0.5 * (_IMW - 1)

    x = to_px(tx)
    y = to_px(ty)
    x0f = jnp.floor(x)
    y0f = jnp.floor(y)
    wx1 = x - x0f
    wx0 = 1.0 - wx1
    wy1 = y - y0f
    wy0 = 1.0 - wy1
    x0 = jnp.clip(x0f.astype(jnp.int32), 0, _IMW - 1)
    y0 = jnp.clip(y0f.astype(jnp.int32), 0, _IMH - 1)
    x1 = jnp.minimum(x0 + 1, _IMW - 1)
    y1 = jnp.minimum(y0 + 1, _IMH - 1)
    idx_ref[0, 0] = y0 * _IMW + x0
    idx_ref[1, 0] = y0 * _IMW + x1
    idx_ref[2, 0] = y1 * _IMW + x0
    idx_ref[3, 0] = y1 * _IMW + x1
    w_ref[0, 0] = wx0 * wy0
    w_ref[1, 0] = wx1 * wy0
    w_ref[2, 0] = wx0 * wy1
    w_ref[3, 0] = wx1 * wy1


_ROWS_PER_TC_BLOCK = 32
_coord_call = pl.pallas_call(
    _coord_body,
    grid=(_B, _H // _ROWS_PER_TC_BLOCK),
    in_specs=[pl.BlockSpec((1, 4, _ROWS_PER_TC_BLOCK, _W),
                           lambda b, r: (b, 0, r, 0))],
    out_specs=[pl.BlockSpec((4, 1, _ROWS_PER_TC_BLOCK, _W),
                            lambda b, r: (0, b, r, 0)),
               pl.BlockSpec((4, 1, _ROWS_PER_TC_BLOCK, _W),
                            lambda b, r: (0, b, r, 0))],
    out_shape=[jax.ShapeDtypeStruct((4, _B, _H, _W), jnp.int32),
               jax.ShapeDtypeStruct((4, _B, _H, _W), jnp.float32)],
)


def _sc_body(img, idxs, ws, out, idx_v, w_v, r_v, out_v, gsem):
    wid = lax.axis_index("s") * 2 + lax.axis_index("c")

    def chunk_body(t, carry):
        base = wid * _PERW + t * _P
        for c in range(4):
            pltpu.sync_copy(idxs.at[c, pl.ds(base, _P)],
                            idx_v.at[pl.ds(c * _P, _P)])
            pltpu.sync_copy(ws.at[c, pl.ds(base, _P)],
                            w_v.at[pl.ds(c * _P, _P)])
        cps = []
        for c in range(4):
            for ch in range(3):
                cps.append(pltpu.async_copy(
                    img.at[pl.ds(ch * _NPIX, _NPIX)]
                       .at[idx_v.at[pl.ds(c * _P, _P)]],
                    r_v.at[pl.ds((c * 3 + ch) * _P, _P)],
                    gsem))
        for cp in cps:
            cp.wait()

        def j_body(j, carry2):
            off = j * 16
            wvs = [w_v[pl.ds(c * _P + off, 16)] for c in range(4)]
            for ch in range(3):
                acc = None
                for c in range(4):
                    term = wvs[c] * r_v[pl.ds((c * 3 + ch) * _P + off, 16)]
                    acc = term if acc is None else acc + term
                out_v[pl.ds(ch * _P + off, 16)] = acc
            return carry2

        lax.fori_loop(0, _P // 16, j_body, 0, unroll=2)

        b = wid // 16
        inb = base - b * _HW
        for ch in range(3):
            pltpu.sync_copy(out_v.at[pl.ds(ch * _P, _P)],
                            out.at[pl.ds((b * 3 + ch) * _HW + inb, _P)])
        return carry

    lax.fori_loop(0, _CHUNKS, chunk_body, 0)


def _sc_call(img, idxs, ws):
    mesh = plsc.VectorSubcoreMesh(core_axis_name="c", subcore_axis_name="s")
    f = pl.kernel(
        _sc_body,
        out_type=jax.ShapeDtypeStruct((_B * 3 * _HW,), jnp.float32),
        mesh=mesh,
        scratch_types=[
            pltpu.VMEM((4 * _P,), jnp.int32),
            pltpu.VMEM((4 * _P,), jnp.float32),
            pltpu.VMEM((12 * _P,), jnp.float32),
            pltpu.VMEM((3 * _P,), jnp.float32),
            pltpu.SemaphoreType.DMA,
        ],
    )
    return f(img, idxs, ws)


def kernel(inputs, ref_img):
    img = ref_img.reshape(3 * _NPIX)
    idx4, w4 = _coord_call(inputs)
    idxs = idx4.reshape(4, _NPOS)
    ws = w4.reshape(4, _NPOS)
    outflat = _sc_call(img, idxs, ws)
    return outflat.reshape(_B, 3, _H, _W)


# speedup vs baseline: 1.4730x; 1.3080x over previous
import math

import jax
import jax.numpy as jnp
from jax import lax
from jax.experimental import pallas as pl
from jax.experimental.pallas import tpu as pltpu
from jax.experimental.pallas import tpu_sc as plsc

_H = 512
_W = 512
_B = 2
_IMH = 2048
_IMW = 2048
_NPOS = _B * _H * _W
_NPIX = _IMH * _IMW
_HW = _H * _W

_NW = 32
_PERW = _NPOS // _NW
_P = 2048
_CHUNKS = _PERW // _P

_PI = math.pi
_RB = 32
_BLK = _RB * _W


def _coord_body(in_ref, i0, i1, i2, i3, w0, w1, w2, w3):
    idx_refs = [i0, i1, i2, i3]
    w_refs = [w0, w1, w2, w3]
    a0 = -in_ref[0, 0]
    a1 = -in_ref[0, 1]
    a2 = -in_ref[0, 2]
    a3 = -in_ref[0, 3]
    ty = jnp.arctan2(a1, a0)
    tx = jnp.arctan2(a3, a2)

    def to_px(t):
        ic = (t + _PI) / (2.0 * _PI)
        ic = -1.0 + 2.0 * ic
        return (ic + 1.0) * 0.5 * (_IMW - 1)

    x = to_px(tx)
    y = to_px(ty)
    x0f = jnp.floor(x)
    y0f = jnp.floor(y)
    wx1 = x - x0f
    wx0 = 1.0 - wx1
    wy1 = y - y0f
    wy0 = 1.0 - wy1
    x0 = jnp.clip(x0f.astype(jnp.int32), 0, _IMW - 1)
    y0 = jnp.clip(y0f.astype(jnp.int32), 0, _IMH - 1)
    x1 = jnp.minimum(x0 + 1, _IMW - 1)
    y1 = jnp.minimum(y0 + 1, _IMH - 1)
    idxs = [y0 * _IMW + x0, y0 * _IMW + x1, y1 * _IMW + x0, y1 * _IMW + x1]
    wsv = [wx0 * wy0, wx1 * wy0, wx0 * wy1, wx1 * wy1]
    for c in range(4):
        idx_refs[c][...] = idxs[c].reshape(_BLK)
        w_refs[c][...] = wsv[c].reshape(_BLK)


_spec1d = pl.BlockSpec((_BLK,), lambda b, r: (b * (_H // _RB) + r,))
_coord_call = pl.pallas_call(
    _coord_body,
    grid=(_B, _H // _RB),
    in_specs=[pl.BlockSpec((1, 4, _RB, _W), lambda b, r: (b, 0, r, 0))],
    out_specs=[_spec1d] * 8,
    out_shape=[jax.ShapeDtypeStruct((_NPOS,), jnp.int32)] * 4
              + [jax.ShapeDtypeStruct((_NPOS,), jnp.float32)] * 4,
)


def _sc_body(img, i0, i1, i2, i3, w0, w1, w2, w3, out,
             idx_v0, idx_v1, w_v0, w_v1, r_v0, r_v1, o_v0, o_v1,
             isem, gsem, osem):
    idx_args = [i0, i1, i2, i3]
    w_args = [w0, w1, w2, w3]
    idx_bufs = [idx_v0, idx_v1]
    w_bufs = [w_v0, w_v1]
    r_bufs = [r_v0, r_v1]
    o_bufs = [o_v0, o_v1]
    wid = lax.axis_index("s") * 2 + lax.axis_index("c")
    b = wid // 16

    def issue_idxw(t, k):
        base = wid * _PERW + t * _P
        hs = []
        for c in range(4):
            hs.append(pltpu.async_copy(idx_args[c].at[pl.ds(base, _P)],
                                       idx_bufs[k].at[pl.ds(c * _P, _P)],
                                       isem))
            hs.append(pltpu.async_copy(w_args[c].at[pl.ds(base, _P)],
                                       w_bufs[k].at[pl.ds(c * _P, _P)],
                                       isem))
        return hs

    def issue_gathers(k):
        hs = []
        for c in range(4):
            for ch in range(3):
                hs.append(pltpu.async_copy(
                    img.at[pl.ds(ch * _NPIX, _NPIX)]
                       .at[idx_bufs[k].at[pl.ds(c * _P, _P)]],
                    r_bufs[k].at[pl.ds((c * 3 + ch) * _P, _P)],
                    gsem))
        return hs

    def combine(k):
        w_v, r_v, out_v = w_bufs[k], r_bufs[k], o_bufs[k]

        def j_body(j, carry2):
            off = j * 16
            wvs = [w_v[pl.ds(c * _P + off, 16)] for c in range(4)]
            for ch in range(3):
                acc = None
                for c in range(4):
                    term = wvs[c] * r_v[pl.ds((c * 3 + ch) * _P + off, 16)]
                    acc = term if acc is None else acc + term
                out_v[pl.ds(ch * _P + off, 16)] = acc
            return carry2

        lax.fori_loop(0, _P // 16, j_body, 0, unroll=2)

    def issue_outwrite(t, k):
        base = wid * _PERW + t * _P
        inb = base - b * _HW
        hs = []
        for ch in range(3):
            hs.append(pltpu.async_copy(
                o_bufs[k].at[pl.ds(ch * _P, _P)],
                out.at[pl.ds((b * 3 + ch) * _HW + inb, _P)],
                osem))
        return hs

    h_iw = [None] * (_CHUNKS + 2)
    h_g = [None] * _CHUNKS
    h_o = [None] * _CHUNKS

    h_iw[0] = issue_idxw(0, 0)
    for h in h_iw[0]:
        h.wait()
    h_g[0] = issue_gathers(0)
    h_iw[1] = issue_idxw(1, 1)

    for t in range(_CHUNKS):
        k = t % 2
        for h in h_g[t]:
            h.wait()
        if t + 1 < _CHUNKS:
            for h in h_iw[t + 1]:
                h.wait()
            h_g[t + 1] = issue_gathers(1 - k)
        if t >= 2:
            for h in h_o[t - 2]:
                h.wait()
        combine(k)
        h_o[t] = issue_outwrite(t, k)
        if t + 2 < _CHUNKS:
            h_iw[t + 2] = issue_idxw(t + 2, k)
    for h in h_o[_CHUNKS - 2]:
        h.wait()
    for h in h_o[_CHUNKS - 1]:
        h.wait()


def _sc_call(img, idxs, ws):
    mesh = plsc.VectorSubcoreMesh(core_axis_name="c", subcore_axis_name="s")
    f = pl.kernel(
        _sc_body,
        out_type=jax.ShapeDtypeStruct((_B * 3 * _HW,), jnp.float32),
        mesh=mesh,
        scratch_types=[
            pltpu.VMEM((4 * _P,), jnp.int32),
            pltpu.VMEM((4 * _P,), jnp.int32),
            pltpu.VMEM((4 * _P,), jnp.float32),
            pltpu.VMEM((4 * _P,), jnp.float32),
            pltpu.VMEM((12 * _P,), jnp.float32),
            pltpu.VMEM((12 * _P,), jnp.float32),
            pltpu.VMEM((3 * _P,), jnp.float32),
            pltpu.VMEM((3 * _P,), jnp.float32),
            pltpu.SemaphoreType.DMA,
            pltpu.SemaphoreType.DMA,
            pltpu.SemaphoreType.DMA,
        ],
    )
    return f(img, *idxs, *ws)


def kernel(inputs, ref_img):
    img = ref_img.reshape(3 * _NPIX)
    outs = _coord_call(inputs)
    idxs = outs[:4]
    ws = outs[4:]
    outflat = _sc_call(img, idxs, ws)
    return outflat.reshape(_B, 3, _H, _W)
